# Initial kernel scaffold; baseline (speedup 1.0000x reference)
import functools

import jax
import jax.numpy as jnp
from jax import lax
from jax.experimental import pallas as pl
from jax.experimental.pallas import tpu as pltpu
from jax.experimental.pallas import tpu_sc as plsc

NC = 2
NS = 16
NW = NC * NS
K = 128
DC = 8


def _sc_mesh():
  return plsc.VectorSubcoreMesh(
      core_axis_name="c", subcore_axis_name="s", num_cores=NC,
      num_subcores=NS)


def _make_deg(n_pad, chunks):
  rpt = n_pad // NS
  nz = rpt // K

  @functools.partial(
      pl.kernel,
      out_type=jax.ShapeDtypeStruct((NC, n_pad, DC), jnp.float32),
      mesh=_sc_mesh(),
      scratch_types=[
          pltpu.VMEM((chunks, K), jnp.int32),
          pltpu.VMEM((K, DC), jnp.float32),
          pltpu.VMEM_SHARED((n_pad, DC), jnp.float32),
      ],
  )
  def deg_kernel(dst_hbm, zeros_hbm, ones_hbm, out_hbm, dst_v, val_v, acc):
    cid = lax.axis_index("c")
    sid = lax.axis_index("s")
    wid = sid * NC + cid
    pltpu.sync_copy(dst_hbm.at[wid], dst_v)
    pltpu.sync_copy(zeros_hbm, val_v)
    for t in range(nz):
      pltpu.sync_copy(val_v, acc.at[pl.ds(sid * rpt + t * K, K)])
    plsc.subcore_barrier()
    pltpu.sync_copy(ones_hbm, val_v)

    def body(j, carry):
      pltpu.sync_copy(val_v, acc.at[dst_v.at[j]], add=True)
      return carry

    lax.fori_loop(0, chunks, body, 0)
    plsc.subcore_barrier()
    for t in range(nz):
      sl = pl.ds(sid * rpt + t * K, K)
      pltpu.sync_copy(acc.at[sl], out_hbm.at[cid, sl])

  return deg_kernel


def _make_agg(n, n_pad, chunks, d):
  rpt = n_pad // NS
  nz = rpt // K

  @functools.partial(
      pl.kernel,
      out_type=jax.ShapeDtypeStruct((NC, n_pad, d), jnp.float32),
      mesh=_sc_mesh(),
      scratch_types=[
          pltpu.VMEM((chunks, K), jnp.int32),
          pltpu.VMEM((chunks, K), jnp.int32),
          pltpu.VMEM((K, d), jnp.float32),
          pltpu.VMEM((K, d), jnp.float32),
          pltpu.VMEM_SHARED((n_pad, d), jnp.float32),
          pltpu.SemaphoreType.DMA,
          pltpu.SemaphoreType.DMA,
      ],
  )
  def agg_kernel(y_hbm, src_hbm, dst_hbm, zeros_hbm, out_hbm,
                 src_v, dst_v, buf0, buf1, acc, sem0, sem1):
    cid = lax.axis_index("c")
    sid = lax.axis_index("s")
    wid = sid * NC + cid
    pltpu.sync_copy(src_hbm.at[wid], src_v)
    pltpu.sync_copy(dst_hbm.at[wid], dst_v)
    pltpu.sync_copy(zeros_hbm, buf0)
    for t in range(nz):
      pltpu.sync_copy(buf0, acc.at[pl.ds(sid * rpt + t * K, K)])
    plsc.subcore_barrier()

    def body(j, carry):
      c0 = pltpu.async_copy(y_hbm.at[src_v.at[2 * j]], buf0, sem0)
      c1 = pltpu.async_copy(y_hbm.at[src_v.at[2 * j + 1]], buf1, sem1)
      c0.wait()
      pltpu.sync_copy(buf0, acc.at[dst_v.at[2 * j]], add=True)
      c1.wait()
      pltpu.sync_copy(buf1, acc.at[dst_v.at[2 * j + 1]], add=True)
      return carry

    lax.fori_loop(0, chunks // 2, body, 0)
    plsc.subcore_barrier()
    for t in range(nz):
      sl = pl.ds(sid * rpt + t * K, K)
      pltpu.sync_copy(acc.at[sl], out_hbm.at[cid, sl])

  return agg_kernel


def _tc_scale_matmul(x, w, counts, bm):
  n, dk = x.shape
  dout = w.shape[1]

  def body(x_ref, w_ref, c_ref, y_ref):
    cnt = c_ref[0, :, 0:1] + c_ref[1, :, 0:1] + 1.0
    dinv = lax.rsqrt(cnt)
    y_ref[...] = jnp.dot(x_ref[...], w_ref[...],
                         preferred_element_type=jnp.float32) * dinv

  return pl.pallas_call(
      body,
      grid=(n // bm,),
      in_specs=[
          pl.BlockSpec((bm, dk), lambda i: (i, 0)),
          pl.BlockSpec((dk, dout), lambda i: (0, 0)),
          pl.BlockSpec((NC, bm, DC), lambda i: (0, i, 0)),
      ],
      out_specs=pl.BlockSpec((bm, dout), lambda i: (i, 0)),
      out_shape=jax.ShapeDtypeStruct((n, dout), jnp.float32),
  )(x, w, counts)


def _tc_combine_matmul(p, y1, counts, b1, w2, bm):
  n, dk = y1.shape
  dout = w2.shape[1]

  def body(p_ref, y1_ref, c_ref, b_ref, w_ref, y2_ref):
    cnt = c_ref[0, :, 0:1] + c_ref[1, :, 0:1] + 1.0
    dinv = lax.rsqrt(cnt)
    h = (p_ref[0] + p_ref[1] + y1_ref[...]) * dinv + b_ref[...]
    h = jnp.maximum(h, 0.0)
    y2_ref[...] = jnp.dot(h, w_ref[...],
                          preferred_element_type=jnp.float32) * dinv

  return pl.pallas_call(
      body,
      grid=(n // bm,),
      in_specs=[
          pl.BlockSpec((NC, bm, dk), lambda i: (0, i, 0)),
          pl.BlockSpec((bm, dk), lambda i: (i, 0)),
          pl.BlockSpec((NC, bm, DC), lambda i: (0, i, 0)),
          pl.BlockSpec((1, dk), lambda i: (0, 0)),
          pl.BlockSpec((dk, dout), lambda i: (0, 0)),
      ],
      out_specs=pl.BlockSpec((bm, dout), lambda i: (i, 0)),
      out_shape=jax.ShapeDtypeStruct((n, dout), jnp.float32),
  )(p, y1, counts, b1, w2)


def _tc_combine(p, y2, counts, b2, bm):
  n, dout = y2.shape

  def body(p_ref, y2_ref, c_ref, b_ref, o_ref):
    cnt = c_ref[0, :, 0:1] + c_ref[1, :, 0:1] + 1.0
    dinv = lax.rsqrt(cnt)
    o_ref[...] = (p_ref[0] + p_ref[1] + y2_ref[...]) * dinv + b_ref[...]

  return pl.pallas_call(
      body,
      grid=(n // bm,),
      in_specs=[
          pl.BlockSpec((NC, bm, dout), lambda i: (0, i, 0)),
          pl.BlockSpec((bm, dout), lambda i: (i, 0)),
          pl.BlockSpec((NC, bm, DC), lambda i: (0, i, 0)),
          pl.BlockSpec((1, dout), lambda i: (0, 0)),
      ],
      out_specs=pl.BlockSpec((bm, dout), lambda i: (i, 0)),
      out_shape=jax.ShapeDtypeStruct((n, dout), jnp.float32),
  )(p, y2, counts, b2)


@jax.jit
def kernel(x, edge_index, W1, b1, W2, b2):
  n = x.shape[0]
  e = edge_index.shape[1]
  d1 = W1.shape[1]
  d2 = W2.shape[1]

  n_pad = -(-n // (NS * K)) * NS * K
  chunks = -(-e // (NW * K))
  chunks += chunks % 2
  e_pad = NW * chunks * K

  src = edge_index[0].astype(jnp.int32)
  dst = edge_index[1].astype(jnp.int32)
  pad = e_pad - e
  src_r = jnp.concatenate([src, jnp.zeros((pad,), jnp.int32)])
  dst_r = jnp.concatenate([dst, jnp.full((pad,), n, jnp.int32)])
  src_r = src_r.reshape(NW, chunks, K)
  dst_r = dst_r.reshape(NW, chunks, K)

  zeros_c = jnp.zeros((K, DC), jnp.float32)
  ones_c = jnp.ones((K, DC), jnp.float32)
  zeros_1 = jnp.zeros((K, d1), jnp.float32)
  zeros_2 = jnp.zeros((K, d2), jnp.float32)
  b1r = b1.reshape(1, d1)
  b2r = b2.reshape(1, d2)

  bm = 1000 if n % 1000 == 0 else 8 * (n // 8)

  counts = _make_deg(n_pad, chunks)(dst_r, zeros_c, ones_c)
  y1 = _tc_scale_matmul(x, W1, counts, bm)
  p1 = _make_agg(n, n_pad, chunks, d1)(y1, src_r, dst_r, zeros_1)
  y2 = _tc_combine_matmul(p1, y1, counts, b1r, W2, bm)
  p2 = _make_agg(n, n_pad, chunks, d2)(y2, src_r, dst_r, zeros_2)
  out = _tc_combine(p2, y2, counts, b2r, bm)
  return out


# speedup vs baseline: 18.1605x; 18.1605x over previous
import functools

import jax
import jax.numpy as jnp
from jax import lax
from jax.experimental import pallas as pl
from jax.experimental.pallas import tpu as pltpu
from jax.experimental.pallas import tpu_sc as plsc

NC = 2
NS = 16
NW = NC * NS
K = 128
DC = 8


def _sc_mesh():
  return plsc.VectorSubcoreMesh(
      core_axis_name="c", subcore_axis_name="s", num_cores=NC,
      num_subcores=NS)


def _make_deg(n_pad, chunks):
  rpt = n_pad // NS
  nz = rpt // K

  @functools.partial(
      pl.kernel,
      out_type=jax.ShapeDtypeStruct((NC, n_pad, DC), jnp.float32),
      mesh=_sc_mesh(),
      compiler_params=pltpu.CompilerParams(use_tc_tiling_on_sc=False),
      scratch_types=[
          pltpu.VMEM((chunks, K), jnp.int32),
          pltpu.VMEM((K, DC), jnp.float32),
          pltpu.VMEM_SHARED((n_pad, DC), jnp.float32),
      ],
  )
  def deg_kernel(dst_hbm, zeros_hbm, ones_hbm, out_hbm, dst_v, val_v, acc):
    cid = lax.axis_index("c")
    sid = lax.axis_index("s")
    wid = sid * NC + cid
    pltpu.sync_copy(dst_hbm.at[wid], dst_v)
    pltpu.sync_copy(zeros_hbm, val_v)
    for t in range(nz):
      pltpu.sync_copy(val_v, acc.at[pl.ds(sid * rpt + t * K, K)])
    plsc.subcore_barrier()
    pltpu.sync_copy(ones_hbm, val_v)

    def body(j, carry):
      pltpu.sync_copy(val_v, acc.at[dst_v.at[j]], add=True)
      return carry

    lax.fori_loop(0, chunks, body, 0)
    plsc.subcore_barrier()

    @pl.when(cid == 0)
    def _():
      for t in range(nz):
        sl = pl.ds(sid * rpt + t * K, K)
        pltpu.sync_copy(acc.at[sl], out_hbm.at[0, sl])

    @pl.when(cid == 1)
    def _():
      for t in range(nz):
        sl = pl.ds(sid * rpt + t * K, K)
        pltpu.sync_copy(acc.at[sl], out_hbm.at[1, sl])

  return deg_kernel


def _make_agg(n_pad, chunks, dh):
  rpt = n_pad // NS
  nz = rpt // K

  @functools.partial(
      pl.kernel,
      out_type=jax.ShapeDtypeStruct((NC, n_pad, dh), jnp.float32),
      mesh=_sc_mesh(),
      compiler_params=pltpu.CompilerParams(use_tc_tiling_on_sc=False),
      scratch_types=[
          pltpu.VMEM((chunks, K), jnp.int32),
          pltpu.VMEM((chunks, K), jnp.int32),
          pltpu.VMEM((K, dh), jnp.float32),
          pltpu.VMEM((K, dh), jnp.float32),
          pltpu.VMEM_SHARED((n_pad, dh), jnp.float32),
          pltpu.SemaphoreType.DMA,
          pltpu.SemaphoreType.DMA,
      ],
  )
  def agg_kernel(y_hbm, src_hbm, dst_hbm, zeros_hbm, out_hbm,
                 src_v, dst_v, buf0, buf1, acc, sem0, sem1):
    cid = lax.axis_index("c")
    sid = lax.axis_index("s")
    pltpu.sync_copy(src_hbm.at[sid], src_v)
    pltpu.sync_copy(dst_hbm.at[sid], dst_v)
    pltpu.sync_copy(zeros_hbm, buf0)
    for t in range(nz):
      pltpu.sync_copy(buf0, acc.at[pl.ds(sid * rpt + t * K, K)])
    plsc.subcore_barrier()

    def run(y_half, out_half):
      def body(j, carry):
        c0 = pltpu.async_copy(y_half.at[src_v.at[2 * j]], buf0, sem0)
        c1 = pltpu.async_copy(y_half.at[src_v.at[2 * j + 1]], buf1, sem1)
        c0.wait()
        pltpu.sync_copy(buf0, acc.at[dst_v.at[2 * j]], add=True)
        c1.wait()
        pltpu.sync_copy(buf1, acc.at[dst_v.at[2 * j + 1]], add=True)
        return carry

      lax.fori_loop(0, chunks // 2, body, 0)
      plsc.subcore_barrier()
      for t in range(nz):
        sl = pl.ds(sid * rpt + t * K, K)
        pltpu.sync_copy(acc.at[sl], out_half.at[sl])

    @pl.when(cid == 0)
    def _():
      run(y_hbm.at[0], out_hbm.at[0])

    @pl.when(cid == 1)
    def _():
      run(y_hbm.at[1], out_hbm.at[1])

  return agg_kernel


def _tc_scale_matmul(x, w, counts, bm):
  n, dk = x.shape
  dout = w.shape[1]
  dh = dout // NC

  def body(x_ref, w_ref, c_ref, y_ref):
    cnt = c_ref[0, :, 0:1] + c_ref[1, :, 0:1] + 1.0
    dinv = lax.rsqrt(cnt)
    y = jnp.dot(x_ref[...], w_ref[...],
                preferred_element_type=jnp.float32) * dinv
    y_ref[0] = y[:, :dh]
    y_ref[1] = y[:, dh:]

  return pl.pallas_call(
      body,
      grid=(n // bm,),
      in_specs=[
          pl.BlockSpec((bm, dk), lambda i: (i, 0)),
          pl.BlockSpec((dk, dout), lambda i: (0, 0)),
          pl.BlockSpec((NC, bm, DC), lambda i: (0, i, 0)),
      ],
      out_specs=pl.BlockSpec((NC, bm, dh), lambda i: (0, i, 0)),
      out_shape=jax.ShapeDtypeStruct((NC, n, dh), jnp.float32),
  )(x, w, counts)


def _tc_combine_matmul(p, y1, counts, b1, w2, bm, dh):
  n = y1.shape[1]
  dk = NC * y1.shape[2]
  dout = w2.shape[1]
  dh_in = y1.shape[2]

  def body(p_ref, y1_ref, c_ref, b_ref, w_ref, y2_ref):
    cnt = c_ref[0, :, 0:1] + c_ref[1, :, 0:1] + 1.0
    dinv = lax.rsqrt(cnt)
    agg = jnp.concatenate([p_ref[0], p_ref[1]], axis=-1)
    y1f = jnp.concatenate([y1_ref[0], y1_ref[1]], axis=-1)
    h = (agg + y1f) * dinv + b_ref[...]
    h = jnp.maximum(h, 0.0)
    y2 = jnp.dot(h, w_ref[...], preferred_element_type=jnp.float32) * dinv
    y2 = jnp.concatenate(
        [y2, jnp.zeros((y2.shape[0], NC * dh - dout), jnp.float32)], axis=-1)
    y2_ref[0] = y2[:, :dh]
    y2_ref[1] = y2[:, dh:]

  return pl.pallas_call(
      body,
      grid=(n // bm,),
      in_specs=[
          pl.BlockSpec((NC, bm, dh_in), lambda i: (0, i, 0)),
          pl.BlockSpec((NC, bm, dh_in), lambda i: (0, i, 0)),
          pl.BlockSpec((NC, bm, DC), lambda i: (0, i, 0)),
          pl.BlockSpec((1, dk), lambda i: (0, 0)),
          pl.BlockSpec((dk, dout), lambda i: (0, 0)),
      ],
      out_specs=pl.BlockSpec((NC, bm, dh), lambda i: (0, i, 0)),
      out_shape=jax.ShapeDtypeStruct((NC, n, dh), jnp.float32),
  )(p, y1, counts, b1, w2)


def _tc_combine(p, y2, counts, b2, bm, dout):
  n = y2.shape[1]
  dh = y2.shape[2]

  def body(p_ref, y2_ref, c_ref, b_ref, o_ref):
    cnt = c_ref[0, :, 0:1] + c_ref[1, :, 0:1] + 1.0
    dinv = lax.rsqrt(cnt)
    agg = jnp.concatenate([p_ref[0], p_ref[1]], axis=-1)
    y2f = jnp.concatenate([y2_ref[0], y2_ref[1]], axis=-1)
    o_ref[...] = ((agg + y2f) * dinv)[:, :dout] + b_ref[...]

  return pl.pallas_call(
      body,
      grid=(n // bm,),
      in_specs=[
          pl.BlockSpec((NC, bm, dh), lambda i: (0, i, 0)),
          pl.BlockSpec((NC, bm, dh), lambda i: (0, i, 0)),
          pl.BlockSpec((NC, bm, DC), lambda i: (0, i, 0)),
          pl.BlockSpec((1, dout), lambda i: (0, 0)),
      ],
      out_specs=pl.BlockSpec((bm, dout), lambda i: (i, 0)),
      out_shape=jax.ShapeDtypeStruct((n, dout), jnp.float32),
  )(p, y2, counts, b2)


@jax.jit
def kernel(x, edge_index, W1, b1, W2, b2):
  n = x.shape[0]
  e = edge_index.shape[1]
  d1 = W1.shape[1]
  d2 = W2.shape[1]
  assert d1 % NC == 0 and d2 % NC == 0

  n_pad = -(-n // (NS * K)) * NS * K

  src = edge_index[0].astype(jnp.int32)
  dst = edge_index[1].astype(jnp.int32)

  chunks_d = -(-e // (NW * K))
  pad_d = NW * chunks_d * K - e
  dst_d = jnp.concatenate([dst, jnp.full((pad_d,), n, jnp.int32)])
  dst_d = dst_d.reshape(NW, chunks_d, K)

  chunks_a = -(-e // (NS * K))
  chunks_a += chunks_a % 2
  pad_a = NS * chunks_a * K - e
  src_a = jnp.concatenate([src, jnp.zeros((pad_a,), jnp.int32)])
  dst_a = jnp.concatenate([dst, jnp.full((pad_a,), n, jnp.int32)])
  src_a = src_a.reshape(NS, chunks_a, K)
  dst_a = dst_a.reshape(NS, chunks_a, K)

  zeros_c = jnp.zeros((K, DC), jnp.float32)
  ones_c = jnp.ones((K, DC), jnp.float32)
  dh1 = d1 // NC
  dh2 = -(-d2 // (NC * 8)) * 8
  zeros_1 = jnp.zeros((K, dh1), jnp.float32)
  zeros_2 = jnp.zeros((K, dh2), jnp.float32)
  b1r = b1.reshape(1, d1)
  b2r = b2.reshape(1, d2)

  bm = 1000 if n % 1000 == 0 else 8 * (n // 8)

  counts = _make_deg(n_pad, chunks_d)(dst_d, zeros_c, ones_c)
  y1 = _tc_scale_matmul(x, W1, counts, bm)
  p1 = _make_agg(n_pad, chunks_a, dh1)(y1, src_a, dst_a, zeros_1)
  y2 = _tc_combine_matmul(p1, y1, counts, b1r, W2, bm, dh2)
  p2 = _make_agg(n_pad, chunks_a, dh2)(y2, src_a, dst_a, zeros_2)
  out = _tc_combine(p2, y2, counts, b2r, bm, d2)
  return out
